# baseline (speedup 1.0000x reference)
import functools

import jax
import jax.numpy as jnp
from jax import lax
from jax.experimental import pallas as pl
from jax.experimental.pallas import tpu as pltpu
from jax.experimental.pallas import tpu_sc as plsc

VOCAB = 1_000_000
D = 64
BATCH = 4096
SEQ = 200
SCALE = 8.0

NC, NS, L = 2, 16, 16
NW = NC * NS
BB = BATCH // NW
SBLK = 8
SCH = 2


@functools.partial(
    pl.kernel,
    out_type=jax.ShapeDtypeStruct((SEQ, D, BATCH), jnp.float32),
    mesh=plsc.VectorSubcoreMesh(
        core_axis_name="c", subcore_axis_name="s",
        num_cores=NC, num_subcores=NS),
    scratch_types=[
        pltpu.VMEM((SBLK, BB), jnp.int32),
        pltpu.VMEM((SBLK, BB), jnp.int32),
        pltpu.VMEM((SBLK, BB), jnp.int32),
        pltpu.VMEM((BB, 2 * D), jnp.float32),
        pltpu.VMEM((BB, 2 * D), jnp.float32),
        pltpu.VMEM((D, BB), jnp.float32),
        pltpu.VMEM((D, BB), jnp.float32),
        pltpu.SemaphoreType.DMA,
    ],
    compiler_params=pltpu.CompilerParams(needs_layout_passes=False),
)
def _embed_sc(tok_hbm, tpair_hbm, out_hbm, idx_v, pidx_v, half_v,
              rows0, rows1, st0, st1, sem):
    wid = lax.axis_index("s") * NC + lax.axis_index("c")
    col0 = pl.multiple_of(wid * BB, BB)
    lanes = lax.iota(jnp.int32, L)

    @pl.loop(0, SEQ // SBLK)
    def _block(g):
        pltpu.sync_copy(tok_hbm.at[g, :, pl.ds(col0, BB)], idx_v)
        for r in range(SBLK):
            for q in range(BB // L):
                sl = pl.ds(q * L, L)
                t = idx_v[r, sl]
                pidx_v[r, sl] = lax.shift_right_logical(t, 1)
                half_v[r, sl] = lax.shift_left(jnp.bitwise_and(t, 1), 6)

        @pl.loop(0, SBLK // SCH)
        def _chunk(c):
            r0 = c * SCH
            for sl, (rows, st) in enumerate(((rows0, st0), (rows1, st1))):
                pltpu.async_copy(
                    tpair_hbm.at[pidx_v.at[r0 + sl]], rows, sem).wait()
                for q in range(BB // L):
                    rowv = lanes + q * L
                    colv = half_v[r0 + sl, pl.ds(q * L, L)]

                    @pl.loop(0, D)
                    def _feat(k, rows=rows, st=st, rowv=rowv, colv=colv, q=q):
                        v = plsc.load_gather(rows, [rowv, colv + k])
                        st[k, pl.ds(q * L, L)] = v * SCALE

            s0 = g * SBLK + r0
            pltpu.sync_copy(st0, out_hbm.at[s0, :, pl.ds(col0, BB)])
            pltpu.sync_copy(st1, out_hbm.at[s0 + 1, :, pl.ds(col0, BB)])


def kernel(tokens, table):
    tok3 = tokens.astype(jnp.int32).T.reshape(SEQ // SBLK, SBLK, BATCH)
    tpair = table.reshape(VOCAB // 2, 2 * D)
    res = _embed_sc(tok3, tpair)
    return res.transpose(2, 0, 1)


# speedup vs baseline: 1.1604x; 1.1604x over previous
import functools

import jax
import jax.numpy as jnp
from jax import lax
from jax.experimental import pallas as pl
from jax.experimental.pallas import tpu as pltpu
from jax.experimental.pallas import tpu_sc as plsc

VOCAB = 1_000_000
D = 64
BATCH = 4096
SEQ = 200
SCALE = 8.0

NC, NS, L = 2, 16, 16
NW = NC * NS
BB = BATCH // NW
SBLK = 8


@functools.partial(
    pl.kernel,
    out_type=jax.ShapeDtypeStruct((SEQ, D, BATCH), jnp.float32),
    mesh=plsc.VectorSubcoreMesh(
        core_axis_name="c", subcore_axis_name="s",
        num_cores=NC, num_subcores=NS),
    scratch_types=[
        pltpu.VMEM((SBLK, BB), jnp.int32),
        pltpu.VMEM((SEQ, BB), jnp.int32),
        pltpu.VMEM((SEQ, BB), jnp.int32),
        pltpu.VMEM((BB, 2 * D), jnp.float32),
        pltpu.VMEM((BB, 2 * D), jnp.float32),
        pltpu.VMEM((D, BB), jnp.float32),
        pltpu.VMEM((D, BB), jnp.float32),
        pltpu.SemaphoreType.DMA,
        pltpu.SemaphoreType.DMA,
        pltpu.SemaphoreType.DMA,
        pltpu.SemaphoreType.DMA,
    ],
    compiler_params=pltpu.CompilerParams(needs_layout_passes=False),
)
def _embed_sc(tok_hbm, tpair_hbm, out_hbm, idx_v, pidx_v, fbase_v,
              rows0, rows1, st0, st1, sg0, sg1, so0, so1):
    wid = lax.axis_index("s") * NC + lax.axis_index("c")
    col0 = pl.multiple_of(wid * BB, BB)
    lanes = lax.iota(jnp.int32, L)

    @pl.loop(0, SEQ // SBLK)
    def _block(g):
        pltpu.sync_copy(tok_hbm.at[g, :, pl.ds(col0, BB)], idx_v)
        for r in range(SBLK):
            for q in range(BB // L):
                sl = pl.ds(q * L, L)
                t = idx_v[r, sl]
                pidx_v[g * SBLK + r, sl] = lax.shift_right_logical(t, 1)
                fbase_v[g * SBLK + r, sl] = lax.shift_left(
                    jnp.bitwise_and(t, 1), 6)

    def _gather(s, rows, sem):
        pltpu.async_copy(tpair_hbm.at[pidx_v.at[s]], rows, sem)

    def _transpose(s, rows, st):
        for q in range(BB // L):
            rowv = lanes + q * L
            colv = fbase_v[s, pl.ds(q * L, L)]

            @pl.loop(0, D, unroll=8)
            def _feat(k, rows=rows, st=st, rowv=rowv, colv=colv, q=q):
                v = plsc.load_gather(rows, [rowv, colv + k])
                st[k, pl.ds(q * L, L)] = v * SCALE

    _gather(0, rows0, sg0)
    _gather(1, rows1, sg1)

    @pl.loop(0, SEQ // 2)
    def _pos(ss):
        for p, (rows, st, sg, so) in enumerate(
                ((rows0, st0, sg0, so0), (rows1, st1, sg1, so1))):
            s = 2 * ss + p
            pltpu.make_async_copy(tpair_hbm.at[pidx_v.at[s]], rows, sg).wait()

            @pl.when(ss > 0)
            def _(st=st, so=so, s=s):
                pltpu.make_async_copy(
                    st, out_hbm.at[s - 2, :, pl.ds(col0, BB)], so).wait()

            _transpose(s, rows, st)
            pltpu.async_copy(st, out_hbm.at[s, :, pl.ds(col0, BB)], so)

            @pl.when(ss < SEQ // 2 - 1)
            def _(s=s, rows=rows, sg=sg):
                _gather(s + 2, rows, sg)

    pltpu.make_async_copy(
        st0, out_hbm.at[SEQ - 2, :, pl.ds(col0, BB)], so0).wait()
    pltpu.make_async_copy(
        st1, out_hbm.at[SEQ - 1, :, pl.ds(col0, BB)], so1).wait()


def kernel(tokens, table):
    tok3 = tokens.astype(jnp.int32).T.reshape(SEQ // SBLK, SBLK, BATCH)
    tpair = table.reshape(VOCAB // 2, 2 * D)
    res = _embed_sc(tok3, tpair)
    return res.transpose(2, 0, 1)
